# baseline (speedup 1.0000x reference)
import functools
import jax
import jax.numpy as jnp
from jax import lax
from jax.experimental import pallas as pl
from jax.experimental.pallas import tpu as pltpu
from jax.experimental.pallas import tpu_sc as plsc

N = 10000
NP = 10240
E = 160000
EP = 163840
K = 80
EPT0 = 3520
EPT1 = 6720
RPT = NP // 16
NBLK = 8
RB = NP // NBLK


def _leaky(v):
    return jnp.where(v >= 0, v, 0.2 * v)


def _mm_body(x_ref, w_ref, aa_ref, ab_ref, xh_ref, ta_ref, tb_ref):
    xh = jnp.dot(x_ref[...], w_ref[...], preferred_element_type=jnp.float32)
    xh_ref[...] = xh
    ta_ref[...] = jnp.dot(xh, aa_ref[...], preferred_element_type=jnp.float32)
    tb_ref[...] = jnp.dot(xh, ab_ref[...], preferred_element_type=jnp.float32)


def _tc_project(x, w, aa, ab):
    return pl.pallas_call(
        _mm_body,
        grid=(NBLK,),
        in_specs=[
            pl.BlockSpec((RB, x.shape[1]), lambda i: (i, 0)),
            pl.BlockSpec(w.shape, lambda i: (0, 0)),
            pl.BlockSpec(aa.shape, lambda i: (0, 0)),
            pl.BlockSpec(ab.shape, lambda i: (0, 0)),
        ],
        out_specs=[
            pl.BlockSpec((RB, w.shape[1]), lambda i: (i, 0)),
            pl.BlockSpec((RB, 16), lambda i: (i, 0)),
            pl.BlockSpec((RB, 16), lambda i: (i, 0)),
        ],
        out_shape=[
            jax.ShapeDtypeStruct((NP, w.shape[1]), jnp.float32),
            jax.ShapeDtypeStruct((NP, 16), jnp.float32),
            jax.ShapeDtypeStruct((NP, 16), jnp.float32),
        ],
    )(x, w, aa, ab)


def _make_edge_kernel(head_map):
    mesh = plsc.VectorSubcoreMesh(core_axis_name="c", subcore_axis_name="s")

    @functools.partial(
        pl.kernel,
        mesh=mesh,
        compiler_params=pltpu.CompilerParams(use_tc_tiling_on_sc=False),
        out_type=[
            jax.ShapeDtypeStruct((2, NP, 256), jnp.float32),
            jax.ShapeDtypeStruct((2, NP, 16), jnp.float32),
        ],
        scratch_types=[
            pltpu.VMEM((K,), jnp.int32),
            pltpu.VMEM((K,), jnp.int32),
            pltpu.VMEM((K, 16), jnp.float32),
            pltpu.VMEM((K, 16), jnp.float32),
            pltpu.VMEM((K, 128), jnp.float32),
            pltpu.VMEM((K,), jnp.int32),
            pltpu.VMEM((K,), jnp.int32),
            pltpu.VMEM((K, 16), jnp.float32),
            pltpu.VMEM((K, 16), jnp.float32),
            pltpu.VMEM((K, 128), jnp.float32),
            pltpu.VMEM((K, 16), jnp.float32),
            pltpu.VMEM((K, 16), jnp.float32),
            pltpu.VMEM_SHARED((NP, 128), jnp.float32),
            pltpu.VMEM_SHARED((NP, 16), jnp.float32),
            pltpu.SemaphoreType.DMA,
            pltpu.SemaphoreType.DMA,
            pltpu.SemaphoreType.DMA,
            pltpu.SemaphoreType.DMA,
            pltpu.SemaphoreType.DMA,
            pltpu.SemaphoreType.DMA,
        ],
    )
    def edge_kernel(src_hbm, dst_hbm, ta_hbm, tb_hbm, xlo_hbm, xhi_hbm,
                    p_hbm, s_hbm,
                    srcvA, dstvA, gaA, gbA, xbA,
                    srcvB, dstvB, gaB, gbB, xbB,
                    fbA, fbB, acc, sacc, sA0, sA1, sA2, sB0, sB1, sB2):
        c = lax.axis_index("c")
        s = lax.axis_index("s")
        tile_base = jnp.where(c == 0, s * EPT0, 16 * EPT0 + s * EPT1)
        nchunks = jnp.where(c == 0, EPT0 // K, EPT1 // K)
        r0 = s * RPT
        bufA = (srcvA, dstvA, gaA, gbA, xbA, fbA, sA0, sA1, sA2)
        bufB = (srcvB, dstvB, gaB, gbB, xbB, fbB, sB0, sB1, sB2)

        def zero_xb(k, _):
            for j in range(8):
                xbA[k, pl.ds(16 * j, 16)] = jnp.zeros((16,), jnp.float32)
            return 0

        def zero_ga(k, _):
            gaA[k] = jnp.zeros((16,), jnp.float32)
            return 0

        for half in (0, 1):
            xh_hbm = xlo_hbm if half == 0 else xhi_hbm

            lax.fori_loop(0, K, zero_xb, 0)

            def zrow(i, _):
                pltpu.sync_copy(xbA, acc.at[pl.ds(r0 + i * K, K)])
                return 0
            lax.fori_loop(0, RPT // K, zrow, 0)
            if half == 0:
                lax.fori_loop(0, K, zero_ga, 0)

                def zsrow(i, _):
                    pltpu.sync_copy(gaA, sacc.at[pl.ds(r0 + i * K, K)])
                    return 0
                lax.fori_loop(0, RPT // K, zsrow, 0)
            plsc.subcore_barrier()

            def issue(j, buf):
                srcv, dstv, ga, gb, xb = buf[:5]
                s0, s1, s2 = buf[6:]
                base = pl.multiple_of(tile_base + j * K, 8)
                pltpu.sync_copy(src_hbm.at[pl.ds(base, K)], srcv)
                pltpu.sync_copy(dst_hbm.at[pl.ds(base, K)], dstv)
                pltpu.async_copy(ta_hbm.at[srcv], ga, s0)
                pltpu.async_copy(tb_hbm.at[dstv], gb, s1)
                pltpu.async_copy(xh_hbm.at[srcv], xb, s2)

            def wait(buf):
                srcv, dstv, ga, gb, xb = buf[:5]
                s0, s1, s2 = buf[6:]
                pltpu.make_async_copy(ta_hbm.at[srcv], ga, s0).wait()
                pltpu.make_async_copy(tb_hbm.at[dstv], gb, s1).wait()
                pltpu.make_async_copy(xh_hbm.at[srcv], xb, s2).wait()

            def process(buf):
                srcv, dstv, ga, gb, xb, fb = buf[:6]

                def row(k, _):
                    v = jnp.exp(_leaky(ga[k] + gb[k]))
                    if half == 0:
                        fb[k] = v
                    for j2 in range(8):
                        xb[k, pl.ds(16 * j2, 16)] = (
                            xb[k, pl.ds(16 * j2, 16)] * v[head_map[half][j2]])
                    return 0
                lax.fori_loop(0, K, row, 0, unroll=2)

                pltpu.sync_copy(xb, acc.at[dstv], add=True)
                if half == 0:
                    pltpu.sync_copy(fb, sacc.at[dstv], add=True)

            issue(0, bufA)

            def pipeline(t, _):
                j1 = 2 * t + 1
                wait(bufA)
                issue(j1, bufB)
                process(bufA)
                wait(bufB)

                @pl.when(j1 + 1 < nchunks)
                def _():
                    issue(j1 + 1, bufA)
                process(bufB)
                return 0

            lax.fori_loop(0, nchunks // 2, pipeline, 0)
            plsc.subcore_barrier()

            def frow(i, _):
                row = r0 + i * K
                pltpu.sync_copy(acc.at[pl.ds(row, K)], xbA)
                pltpu.sync_copy(xbA, p_hbm.at[c, pl.ds(row, K),
                                              pl.ds(half * 128, 128)])
                if half == 0:
                    pltpu.sync_copy(sacc.at[pl.ds(row, K)], gaA)
                    pltpu.sync_copy(gaA, s_hbm.at[c, pl.ds(row, K)])
                return 0
            lax.fori_loop(0, RPT // K, frow, 0)
            plsc.subcore_barrier()

    return edge_kernel


HEAD_MAP_L1 = tuple(tuple(h * 4 + j // 2 for j in range(8)) for h in (0, 1))
HEAD_MAP_L2 = ((0,) * 8, (0,) * 8)

_edge_l1 = _make_edge_kernel(HEAD_MAP_L1)
_edge_l2 = _make_edge_kernel(HEAD_MAP_L2)


def _epi1_body(p0_ref, p1_ref, s0_ref, s1_ref, xh_ref, ta_ref, tb_ref,
               e8_ref, b1_ref, w2_ref, aa_ref, ab_ref,
               xh2_ref, t2a_ref, t2b_ref):
    fs = jnp.exp(_leaky(ta_ref[:, :8] + tb_ref[:, :8]))
    e8 = e8_ref[...]
    fse = jnp.dot(fs, e8, preferred_element_type=jnp.float32)
    num = p0_ref[...] + p1_ref[...] + fse * xh_ref[...]
    den = jnp.dot(s0_ref[:, :8] + s1_ref[:, :8] + fs, e8,
                  preferred_element_type=jnp.float32)
    h = num / den + b1_ref[...]
    h = jnp.where(h > 0, h, jnp.exp(jnp.minimum(h, 0.0)) - 1.0)
    xh2 = jnp.dot(h, w2_ref[...], preferred_element_type=jnp.float32)
    xh2_ref[...] = xh2
    t2a_ref[...] = jnp.dot(xh2, aa_ref[...], preferred_element_type=jnp.float32)
    t2b_ref[...] = jnp.dot(xh2, ab_ref[...], preferred_element_type=jnp.float32)


def _tc_epi1(p0, p1, s0, s1, xh1, ta1, tb1, e8, b1, w2, aa, ab):
    row = lambda i: (i, 0)
    full = lambda i: (0, 0)
    return pl.pallas_call(
        _epi1_body,
        grid=(NBLK,),
        in_specs=[
            pl.BlockSpec((RB, 256), row), pl.BlockSpec((RB, 256), row),
            pl.BlockSpec((RB, 16), row), pl.BlockSpec((RB, 16), row),
            pl.BlockSpec((RB, 256), row),
            pl.BlockSpec((RB, 16), row), pl.BlockSpec((RB, 16), row),
            pl.BlockSpec((8, 256), full), pl.BlockSpec((1, 256), full),
            pl.BlockSpec((256, 256), full),
            pl.BlockSpec((256, 16), full), pl.BlockSpec((256, 16), full),
        ],
        out_specs=[
            pl.BlockSpec((RB, 256), row),
            pl.BlockSpec((RB, 16), row),
            pl.BlockSpec((RB, 16), row),
        ],
        out_shape=[
            jax.ShapeDtypeStruct((NP, 256), jnp.float32),
            jax.ShapeDtypeStruct((NP, 16), jnp.float32),
            jax.ShapeDtypeStruct((NP, 16), jnp.float32),
        ],
    )(p0, p1, s0, s1, xh1, ta1, tb1, e8, b1, w2, aa, ab)


def _epi2_body(p0_ref, p1_ref, s0_ref, s1_ref, xh_ref, ta_ref, tb_ref,
               b2_ref, out_ref):
    fs = jnp.exp(_leaky(ta_ref[:, 0:1] + tb_ref[:, 0:1]))
    num = p0_ref[...] + p1_ref[...] + fs * xh_ref[...]
    den = s0_ref[:, 0:1] + s1_ref[:, 0:1] + fs
    out_ref[...] = num / den + b2_ref[...]


def _tc_epi2(p0, p1, s0, s1, xh2, ta, tb, b2):
    row = lambda i: (i, 0)
    full = lambda i: (0, 0)
    return pl.pallas_call(
        _epi2_body,
        grid=(NBLK,),
        in_specs=[
            pl.BlockSpec((RB, 256), row), pl.BlockSpec((RB, 256), row),
            pl.BlockSpec((RB, 16), row), pl.BlockSpec((RB, 16), row),
            pl.BlockSpec((RB, 256), row),
            pl.BlockSpec((RB, 16), row), pl.BlockSpec((RB, 16), row),
            pl.BlockSpec((1, 256), full),
        ],
        out_specs=pl.BlockSpec((RB, 256), row),
        out_shape=jax.ShapeDtypeStruct((NP, 256), jnp.float32),
    )(p0, p1, s0, s1, xh2, ta, tb, b2)


def _att_proj(att, heads, ch):
    a = att.reshape(heads, ch)
    eye = jnp.eye(heads, 16, dtype=jnp.float32)
    return jnp.einsum("hc,hk->hck", a, eye).reshape(heads * ch, 16)


def kernel(x, edge_index, W1, att_src1, att_dst1, b1, W2, att_src2, att_dst2, b2):
    xp = jnp.zeros((NP, 256), jnp.float32).at[:N].set(x)
    src = jnp.full((EP,), N, jnp.int32).at[:E].set(edge_index[0])
    dst = jnp.full((EP,), N, jnp.int32).at[:E].set(edge_index[1])

    a1a = _att_proj(att_src1, 8, 32)
    a1b = _att_proj(att_dst1, 8, 32)
    a2a = _att_proj(att_src2, 1, 256)
    a2b = _att_proj(att_dst2, 1, 256)
    e8 = jnp.repeat(jnp.eye(8, dtype=jnp.float32), 32, axis=1)
    b1r = b1.reshape(1, 256)
    b2r = b2.reshape(1, 256)

    xh1, ta1, tb1 = _tc_project(xp, W1, a1a, a1b)
    p1, s1 = _edge_l1(src, dst, ta1, tb1, xh1[:, :128], xh1[:, 128:])
    xh2, t2a, t2b = _tc_epi1(p1[0], p1[1], s1[0], s1[1], xh1, ta1, tb1,
                             e8, b1r, W2, a2a, a2b)
    p2, s2 = _edge_l2(src, dst, t2a, t2b, xh2[:, :128], xh2[:, 128:])
    out = _tc_epi2(p2[0], p2[1], s2[0], s2[1], xh2, t2a, t2b, b2r)
    return out[:N]


# speedup vs baseline: 1.3232x; 1.3232x over previous
import functools
import jax
import jax.numpy as jnp
from jax import lax
from jax.experimental import pallas as pl
from jax.experimental.pallas import tpu as pltpu
from jax.experimental.pallas import tpu_sc as plsc

N = 10000
NP = 10240
E = 160000
EP = 163840
K = 80
EPT0 = 6720
EPT1 = 3520
RPT = NP // 16
NBLK = 8
RB = NP // NBLK


def _leaky(v):
    return jnp.where(v >= 0, v, 0.2 * v)


def _mm_body(x_ref, w_ref, aa_ref, ab_ref, xh_ref, ta_ref, tb_ref):
    xh = jnp.dot(x_ref[...], w_ref[...], preferred_element_type=jnp.float32)
    xh_ref[...] = xh
    ta_ref[...] = jnp.dot(xh, aa_ref[...], preferred_element_type=jnp.float32)
    tb_ref[...] = jnp.dot(xh, ab_ref[...], preferred_element_type=jnp.float32)


def _tc_project(x, w, aa, ab):
    return pl.pallas_call(
        _mm_body,
        grid=(NBLK,),
        in_specs=[
            pl.BlockSpec((RB, x.shape[1]), lambda i: (i, 0)),
            pl.BlockSpec(w.shape, lambda i: (0, 0)),
            pl.BlockSpec(aa.shape, lambda i: (0, 0)),
            pl.BlockSpec(ab.shape, lambda i: (0, 0)),
        ],
        out_specs=[
            pl.BlockSpec((RB, w.shape[1]), lambda i: (i, 0)),
            pl.BlockSpec((RB, 16), lambda i: (i, 0)),
            pl.BlockSpec((RB, 16), lambda i: (i, 0)),
        ],
        out_shape=[
            jax.ShapeDtypeStruct((NP, w.shape[1]), jnp.float32),
            jax.ShapeDtypeStruct((NP, 16), jnp.float32),
            jax.ShapeDtypeStruct((NP, 16), jnp.float32),
        ],
    )(x, w, aa, ab)


def _make_edge_kernel(head_map):
    mesh = plsc.VectorSubcoreMesh(core_axis_name="c", subcore_axis_name="s")

    @functools.partial(
        pl.kernel,
        mesh=mesh,
        compiler_params=pltpu.CompilerParams(use_tc_tiling_on_sc=False),
        out_type=[
            jax.ShapeDtypeStruct((2, NP, 256), jnp.float32),
            jax.ShapeDtypeStruct((2, NP, 16), jnp.float32),
        ],
        scratch_types=[
            pltpu.VMEM((K,), jnp.int32),
            pltpu.VMEM((K,), jnp.int32),
            pltpu.VMEM((K, 16), jnp.float32),
            pltpu.VMEM((K, 16), jnp.float32),
            pltpu.VMEM((K, 128), jnp.float32),
            pltpu.VMEM((K,), jnp.int32),
            pltpu.VMEM((K,), jnp.int32),
            pltpu.VMEM((K, 16), jnp.float32),
            pltpu.VMEM((K, 16), jnp.float32),
            pltpu.VMEM((K, 128), jnp.float32),
            pltpu.VMEM((K, 16), jnp.float32),
            pltpu.VMEM((K, 16), jnp.float32),
            pltpu.VMEM_SHARED((NP, 128), jnp.float32),
            pltpu.VMEM_SHARED((NP, 16), jnp.float32),
            pltpu.SemaphoreType.DMA,
            pltpu.SemaphoreType.DMA,
            pltpu.SemaphoreType.DMA,
            pltpu.SemaphoreType.DMA,
            pltpu.SemaphoreType.DMA,
            pltpu.SemaphoreType.DMA,
        ],
    )
    def edge_kernel(src_hbm, dst_hbm, ta_hbm, tb_hbm, xlo_hbm, xhi_hbm,
                    p_hbm, s_hbm,
                    srcvA, dstvA, gaA, gbA, xbA,
                    srcvB, dstvB, gaB, gbB, xbB,
                    fbA, fbB, acc, sacc, sA0, sA1, sA2, sB0, sB1, sB2):
        c = lax.axis_index("c")
        s = lax.axis_index("s")
        tile_base = jnp.where(c == 0, s * EPT0, 16 * EPT0 + s * EPT1)
        nchunks = jnp.where(c == 0, EPT0 // K, EPT1 // K)
        r0 = s * RPT
        bufA = (srcvA, dstvA, gaA, gbA, xbA, fbA, sA0, sA1, sA2)
        bufB = (srcvB, dstvB, gaB, gbB, xbB, fbB, sB0, sB1, sB2)

        def zero_xb(k, _):
            for j in range(8):
                xbA[k, pl.ds(16 * j, 16)] = jnp.zeros((16,), jnp.float32)
            return 0

        def zero_ga(k, _):
            gaA[k] = jnp.zeros((16,), jnp.float32)
            return 0

        for half in (0, 1):
            xh_hbm = xlo_hbm if half == 0 else xhi_hbm

            lax.fori_loop(0, K, zero_xb, 0)

            def zrow(i, _):
                pltpu.sync_copy(xbA, acc.at[pl.ds(r0 + i * K, K)])
                return 0
            lax.fori_loop(0, RPT // K, zrow, 0)
            if half == 0:
                lax.fori_loop(0, K, zero_ga, 0)

                def zsrow(i, _):
                    pltpu.sync_copy(gaA, sacc.at[pl.ds(r0 + i * K, K)])
                    return 0
                lax.fori_loop(0, RPT // K, zsrow, 0)
            plsc.subcore_barrier()

            def issue(j, buf):
                srcv, dstv, ga, gb, xb = buf[:5]
                s0, s1, s2 = buf[6:]
                base = pl.multiple_of(tile_base + j * K, 8)
                pltpu.sync_copy(src_hbm.at[pl.ds(base, K)], srcv)
                pltpu.sync_copy(dst_hbm.at[pl.ds(base, K)], dstv)
                pltpu.async_copy(ta_hbm.at[srcv], ga, s0)
                pltpu.async_copy(tb_hbm.at[dstv], gb, s1)
                pltpu.async_copy(xh_hbm.at[srcv], xb, s2)

            def wait(buf):
                srcv, dstv, ga, gb, xb = buf[:5]
                s0, s1, s2 = buf[6:]
                pltpu.make_async_copy(ta_hbm.at[srcv], ga, s0).wait()
                pltpu.make_async_copy(tb_hbm.at[dstv], gb, s1).wait()
                pltpu.make_async_copy(xh_hbm.at[srcv], xb, s2).wait()

            def process(buf):
                srcv, dstv, ga, gb, xb, fb = buf[:6]

                def row(k, _):
                    v = jnp.exp(_leaky(ga[k] + gb[k]))
                    if half == 0:
                        fb[k] = v
                    for j2 in range(8):
                        xb[k, pl.ds(16 * j2, 16)] = (
                            xb[k, pl.ds(16 * j2, 16)] * v[head_map[half][j2]])
                    return 0
                lax.fori_loop(0, K, row, 0, unroll=2)

                pltpu.sync_copy(xb, acc.at[dstv], add=True)
                if half == 0:
                    pltpu.sync_copy(fb, sacc.at[dstv], add=True)

            issue(0, bufA)

            def pipeline(t, _):
                j1 = 2 * t + 1
                wait(bufA)
                issue(j1, bufB)
                process(bufA)
                wait(bufB)

                @pl.when(j1 + 1 < nchunks)
                def _():
                    issue(j1 + 1, bufA)
                process(bufB)
                return 0

            lax.fori_loop(0, nchunks // 2, pipeline, 0)
            plsc.subcore_barrier()

            def frow(i, _):
                row = r0 + i * K
                pltpu.sync_copy(acc.at[pl.ds(row, K)], xbA)
                pltpu.sync_copy(xbA, p_hbm.at[c, pl.ds(row, K),
                                              pl.ds(half * 128, 128)])
                if half == 0:
                    pltpu.sync_copy(sacc.at[pl.ds(row, K)], gaA)
                    pltpu.sync_copy(gaA, s_hbm.at[c, pl.ds(row, K)])
                return 0
            lax.fori_loop(0, RPT // K, frow, 0)
            plsc.subcore_barrier()

    return edge_kernel


HEAD_MAP_L1 = tuple(tuple(h * 4 + j // 2 for j in range(8)) for h in (0, 1))
HEAD_MAP_L2 = ((0,) * 8, (0,) * 8)

_edge_l1 = _make_edge_kernel(HEAD_MAP_L1)
_edge_l2 = _make_edge_kernel(HEAD_MAP_L2)


def _epi1_body(p0_ref, p1_ref, s0_ref, s1_ref, xh_ref, ta_ref, tb_ref,
               e8_ref, b1_ref, w2_ref, aa_ref, ab_ref,
               xh2_ref, t2a_ref, t2b_ref):
    fs = jnp.exp(_leaky(ta_ref[:, :8] + tb_ref[:, :8]))
    e8 = e8_ref[...]
    fse = jnp.dot(fs, e8, preferred_element_type=jnp.float32)
    num = p0_ref[...] + p1_ref[...] + fse * xh_ref[...]
    den = jnp.dot(s0_ref[:, :8] + s1_ref[:, :8] + fs, e8,
                  preferred_element_type=jnp.float32)
    h = num / den + b1_ref[...]
    h = jnp.where(h > 0, h, jnp.exp(jnp.minimum(h, 0.0)) - 1.0)
    xh2 = jnp.dot(h, w2_ref[...], preferred_element_type=jnp.float32)
    xh2_ref[...] = xh2
    t2a_ref[...] = jnp.dot(xh2, aa_ref[...], preferred_element_type=jnp.float32)
    t2b_ref[...] = jnp.dot(xh2, ab_ref[...], preferred_element_type=jnp.float32)


def _tc_epi1(p0, p1, s0, s1, xh1, ta1, tb1, e8, b1, w2, aa, ab):
    row = lambda i: (i, 0)
    full = lambda i: (0, 0)
    return pl.pallas_call(
        _epi1_body,
        grid=(NBLK,),
        in_specs=[
            pl.BlockSpec((RB, 256), row), pl.BlockSpec((RB, 256), row),
            pl.BlockSpec((RB, 16), row), pl.BlockSpec((RB, 16), row),
            pl.BlockSpec((RB, 256), row),
            pl.BlockSpec((RB, 16), row), pl.BlockSpec((RB, 16), row),
            pl.BlockSpec((8, 256), full), pl.BlockSpec((1, 256), full),
            pl.BlockSpec((256, 256), full),
            pl.BlockSpec((256, 16), full), pl.BlockSpec((256, 16), full),
        ],
        out_specs=[
            pl.BlockSpec((RB, 256), row),
            pl.BlockSpec((RB, 16), row),
            pl.BlockSpec((RB, 16), row),
        ],
        out_shape=[
            jax.ShapeDtypeStruct((NP, 256), jnp.float32),
            jax.ShapeDtypeStruct((NP, 16), jnp.float32),
            jax.ShapeDtypeStruct((NP, 16), jnp.float32),
        ],
    )(p0, p1, s0, s1, xh1, ta1, tb1, e8, b1, w2, aa, ab)


def _epi2_body(p0_ref, p1_ref, s0_ref, s1_ref, xh_ref, ta_ref, tb_ref,
               b2_ref, out_ref):
    fs = jnp.exp(_leaky(ta_ref[:, 0:1] + tb_ref[:, 0:1]))
    num = p0_ref[...] + p1_ref[...] + fs * xh_ref[...]
    den = s0_ref[:, 0:1] + s1_ref[:, 0:1] + fs
    out_ref[...] = num / den + b2_ref[...]


def _tc_epi2(p0, p1, s0, s1, xh2, ta, tb, b2):
    row = lambda i: (i, 0)
    full = lambda i: (0, 0)
    return pl.pallas_call(
        _epi2_body,
        grid=(NBLK,),
        in_specs=[
            pl.BlockSpec((RB, 256), row), pl.BlockSpec((RB, 256), row),
            pl.BlockSpec((RB, 16), row), pl.BlockSpec((RB, 16), row),
            pl.BlockSpec((RB, 256), row),
            pl.BlockSpec((RB, 16), row), pl.BlockSpec((RB, 16), row),
            pl.BlockSpec((1, 256), full),
        ],
        out_specs=pl.BlockSpec((RB, 256), row),
        out_shape=jax.ShapeDtypeStruct((NP, 256), jnp.float32),
    )(p0, p1, s0, s1, xh2, ta, tb, b2)


def _att_proj(att, heads, ch):
    a = att.reshape(heads, ch)
    eye = jnp.eye(heads, 16, dtype=jnp.float32)
    return jnp.einsum("hc,hk->hck", a, eye).reshape(heads * ch, 16)


def kernel(x, edge_index, W1, att_src1, att_dst1, b1, W2, att_src2, att_dst2, b2):
    xp = jnp.zeros((NP, 256), jnp.float32).at[:N].set(x)
    src = jnp.full((EP,), N, jnp.int32).at[:E].set(edge_index[0])
    dst = jnp.full((EP,), N, jnp.int32).at[:E].set(edge_index[1])

    a1a = _att_proj(att_src1, 8, 32)
    a1b = _att_proj(att_dst1, 8, 32)
    a2a = _att_proj(att_src2, 1, 256)
    a2b = _att_proj(att_dst2, 1, 256)
    e8 = jnp.repeat(jnp.eye(8, dtype=jnp.float32), 32, axis=1)
    b1r = b1.reshape(1, 256)
    b2r = b2.reshape(1, 256)

    xh1, ta1, tb1 = _tc_project(xp, W1, a1a, a1b)
    p1, s1 = _edge_l1(src, dst, ta1, tb1, xh1[:, :128], xh1[:, 128:])
    xh2, t2a, t2b = _tc_epi1(p1[0], p1[1], s1[0], s1[1], xh1, ta1, tb1,
                             e8, b1r, W2, a2a, a2b)
    p2, s2 = _edge_l2(src, dst, t2a, t2b, xh2[:, :128], xh2[:, 128:])
    out = _tc_epi2(p2[0], p2[1], s2[0], s2[1], xh2, t2a, t2b, b2r)
    return out[:N]
